# baseline (speedup 1.0000x reference)
import functools

import jax
import jax.numpy as jnp
from jax import lax
from jax.experimental import pallas as pl
from jax.experimental.pallas import tpu as pltpu
from jax.experimental.pallas import tpu_sc as plsc

BATCH = 4096
EMBED_DIM = 256
NUM_VERTICES = 8192

_BK = 1024
_NKB = NUM_VERTICES // _BK
_NR = _BK // 8


def _make_sc_gather(dim, batch, dtype):
    info = plsc.get_sparse_core_info()
    nw = info.num_cores * info.num_subcores
    b_per_w = batch // nw
    mesh = plsc.VectorSubcoreMesh(core_axis_name="c", subcore_axis_name="s")

    @functools.partial(
        pl.kernel,
        mesh=mesh,
        out_type=jax.ShapeDtypeStruct((batch, dim), dtype),
        scratch_types=[
            pltpu.VMEM((b_per_w,), jnp.int32),
            pltpu.VMEM((b_per_w, dim), dtype),
            pltpu.SemaphoreType.DMA,
        ],
    )
    def gather(table_hbm, idx_hbm, out_hbm, idx_v, rows_v, sem):
        wid = lax.axis_index("s") * info.num_cores + lax.axis_index("c")
        base = wid * b_per_w
        pltpu.sync_copy(idx_hbm.at[pl.ds(base, b_per_w)], idx_v)
        pltpu.async_copy(table_hbm.at[idx_v], rows_v, sem).wait()
        pltpu.sync_copy(rows_v, out_hbm.at[pl.ds(base, b_per_w)])

    return gather


def _argmin_body(x_ref, v_ref, out_ref, p_ref, mval_ref, midx_ref):
    j = pl.program_id(0)

    @pl.when(j == 0)
    def _():
        mval_ref[...] = jnp.full((BATCH,), -jnp.inf, jnp.float32)
        midx_ref[...] = jnp.zeros((BATCH,), jnp.int32)

    @pl.when(j < _NKB)
    def _():
        x = x_ref[...]
        v = v_ref[...]
        b2h = jnp.sum(v * v, axis=1, keepdims=True) * -0.5
        p_ref[pl.ds((j % 2) * _BK, _BK), :] = lax.dot_general(
            v, x, (((1,), (1,)), ((), ())),
            preferred_element_type=jnp.float32) + b2h

    @pl.when(j > 0)
    def _():
        jj = j - 1
        base = (jj % 2) * _BK

        def step(r, carry):
            acc_v, acc_r = carry
            pv = p_ref[pl.ds(base + r * 8, 8), :]
            m = pv > acc_v
            return jnp.maximum(pv, acc_v), jnp.where(m, r, acc_r)

        acc_v0 = p_ref[pl.ds(base, 8), :]
        acc_r0 = jnp.zeros((8, BATCH), jnp.int32)
        acc_v, acc_r = lax.fori_loop(1, _NR, step, (acc_v0, acc_r0), unroll=8)

        mj = jnp.max(acc_v, axis=0)
        sub = lax.broadcasted_iota(jnp.int32, (8, BATCH), 0)
        cand = jnp.where(acc_v == mj[None, :], acc_r * 8 + sub, NUM_VERTICES)
        kj = jnp.min(cand, axis=0) + jj * _BK

        take = mj > mval_ref[...]
        mval_ref[...] = jnp.where(take, mj, mval_ref[...])
        midx_ref[...] = jnp.where(take, kj, midx_ref[...])

    @pl.when(j == _NKB)
    def _():
        out_ref[0, 0, :] = midx_ref[...]


_argmin_call = pl.pallas_call(
    _argmin_body,
    grid=(_NKB + 1,),
    in_specs=[
        pl.BlockSpec((BATCH, EMBED_DIM), lambda j: (0, 0)),
        pl.BlockSpec((_BK, EMBED_DIM), lambda j: (jnp.minimum(j, _NKB - 1), 0)),
    ],
    out_specs=pl.BlockSpec((1, 1, BATCH), lambda j: (0, 0, 0)),
    out_shape=jax.ShapeDtypeStruct((1, 1, BATCH), jnp.int32),
    scratch_shapes=[
        pltpu.VMEM((2 * _BK, BATCH), jnp.float32),
        pltpu.VMEM((BATCH,), jnp.float32),
        pltpu.VMEM((BATCH,), jnp.int32),
    ],
)

_gather_raw = _make_sc_gather(EMBED_DIM, BATCH, jnp.float32)
_gather_vertex = _make_sc_gather(EMBED_DIM, BATCH, jnp.float32)


def kernel(concept_ids, raw_table, vertex_table):
    ids = concept_ids.astype(jnp.int32)
    initial = _gather_raw(raw_table, ids)
    nearest = _argmin_call(initial, vertex_table).reshape(BATCH)
    final = _gather_vertex(vertex_table, nearest)
    return final
